# Initial kernel scaffold; baseline (speedup 1.0000x reference)
import jax
import jax.numpy as jnp
from jax.experimental import pallas as pl


def kernel(x, target):
    raise NotImplementedError("write your pallas kernel here")


# speedup vs baseline: 4.0276x; 4.0276x over previous
import functools

import jax
import jax.numpy as jnp
import numpy as np
from jax.experimental import pallas as pl
from jax.experimental.pallas import tpu as pltpu

_ANCHORS = np.array(
    [[10, 13], [16, 30], [33, 23], [30, 61], [62, 45], [59, 119],
     [116, 90], [156, 198], [373, 326]], dtype=np.float32)
_NUM_CLASSES = 80
_NUM_ANCHORS = 9
_IMG_DIM = 416.0
_IGNORE_THRES = 0.5
_OBJ_SCALE = 1.0
_NOOBJ_SCALE = 100.0
_B = 16
_G = 52
_S = _G * _G
_C = _NUM_CLASSES + 5
_NT = 64
_STRIDE = _IMG_DIM / _G


def _sig(v):
    return jax.nn.sigmoid(v)


def _clip(p):
    return jnp.clip(p, 1e-7, 1.0 - 1e-7)


def _const_row(vals, shape, axis):
    it = jax.lax.broadcasted_iota(jnp.int32, shape, axis)
    out = jnp.zeros(shape, jnp.float32)
    for i, val in enumerate(vals):
        out = jnp.where(it == i, jnp.float32(val), out)
    return out


def _scalar_pick(vals, idx):
    out = jnp.float32(vals[0])
    for i in range(1, len(vals)):
        out = jnp.where(idx == i, jnp.float32(vals[i]), out)
    return out


def _body(x_ref, tgt_ref, tgtT_ref, out_ref, loss_ref, oh_ref):
    b = pl.program_id(0)
    a = pl.program_id(1)
    b_f = b.astype(jnp.float32)
    a_f = a.astype(jnp.float32)

    v = x_ref[0, 0]

    sg = _sig(v)
    s_iota = jax.lax.broadcasted_iota(jnp.int32, (1, _S), 1)
    gx = jnp.remainder(s_iota, _G).astype(jnp.float32)
    gy = (s_iota // _G).astype(jnp.float32)
    aw = _scalar_pick(list(_ANCHORS[:, 0]), a)
    ah = _scalar_pick(list(_ANCHORS[:, 1]), a)
    row0 = (sg[0:1] + gx) * _STRIDE
    row1 = (sg[1:2] + gy) * _STRIDE
    row2 = jnp.exp(v[2:3]) * aw
    row3 = jnp.exp(v[3:4]) * ah
    res = jnp.concatenate([row0, row1, row2, row3, sg[4:85]], axis=0)
    out_ref[0, 0] = res.T

    pc_all = _clip(sg[4:5])
    base = jnp.sum(-jnp.log(1.0 - pc_all))

    b_row = tgtT_ref[0:1, :]
    lab_row = tgtT_ref[1:2, :]
    cx_row = tgtT_ref[2:3, :] * _G
    cy_row = tgtT_ref[3:4, :] * _G
    gw_row = tgtT_ref[4:5, :] * _G
    gh_row = tgtT_ref[5:6, :] * _G
    b_col = tgt_ref[:, 0:1]
    lab_col = tgt_ref[:, 1:2]
    cx_col = tgt_ref[:, 2:3] * _G
    cy_col = tgt_ref[:, 3:4] * _G
    gw_col = tgt_ref[:, 4:5] * _G
    gh_col = tgt_ref[:, 5:6] * _G

    sa_w = list(_ANCHORS[:, 0] / _STRIDE)
    sa_h = list(_ANCHORS[:, 1] / _STRIDE)

    saw_c = _const_row(sa_w, (_NUM_ANCHORS, 1), 0)
    sah_c = _const_row(sa_h, (_NUM_ANCHORS, 1), 0)
    saw_r = _const_row(sa_w, (1, _NUM_ANCHORS), 1)
    sah_r = _const_row(sa_h, (1, _NUM_ANCHORS), 1)
    inter_ra = jnp.minimum(saw_c, gw_row) * jnp.minimum(sah_c, gh_row)
    iou_ra = inter_ra / (saw_c * sah_c + gw_row * gh_row - inter_ra + 1e-16)
    inter_ar = jnp.minimum(saw_r, gw_col) * jnp.minimum(sah_r, gh_col)
    iou_ar = inter_ar / (saw_r * sah_r + gw_col * gh_col - inter_ar + 1e-16)

    bn_row = jnp.zeros((1, _NT), jnp.float32)
    best_row = iou_ra[0:1, :]
    for j in range(1, _NUM_ANCHORS):
        upd = iou_ra[j:j + 1, :] > best_row
        bn_row = jnp.where(upd, jnp.float32(j), bn_row)
        best_row = jnp.maximum(best_row, iou_ra[j:j + 1, :])
    bn_col = jnp.zeros((_NT, 1), jnp.float32)
    best_col = iou_ar[:, 0:1]
    for j in range(1, _NUM_ANCHORS):
        upd = iou_ar[:, j:j + 1] > best_col
        bn_col = jnp.where(upd, jnp.float32(j), bn_col)
        best_col = jnp.maximum(best_col, iou_ar[:, j:j + 1])

    ioua_row = iou_ra[0:1, :]
    ioua_col = iou_ar[:, 0:1]
    for j in range(1, _NUM_ANCHORS):
        ioua_row = jnp.where(a == j, iou_ra[j:j + 1, :], ioua_row)
        ioua_col = jnp.where(a == j, iou_ar[:, j:j + 1], ioua_col)

    gi_row = jnp.clip(jnp.floor(cx_row), 0.0, _G - 1.0)
    gj_row = jnp.clip(jnp.floor(cy_row), 0.0, _G - 1.0)
    gi_col = jnp.clip(jnp.floor(cx_col), 0.0, _G - 1.0)
    gj_col = jnp.clip(jnp.floor(cy_col), 0.0, _G - 1.0)
    s_row = gj_row * _G + gi_row
    s_col = gj_col * _G + gi_col
    key_row = b_row * jnp.float32(_S) + s_row
    key_col = b_col * jnp.float32(_S) + s_col
    act_row = b_row == b_f

    @pl.when(a == 0)
    def _build_oh():
        it0 = jax.lax.broadcasted_iota(jnp.int32, (_S, _NT), 0)
        s_b = jnp.broadcast_to(s_row.astype(jnp.int32), (_S, _NT))
        actb = jnp.broadcast_to(act_row, (_S, _NT))
        oh_ref[...] = jnp.where((it0 == s_b) & actb, 1.0, 0.0)

    g = jax.lax.dot_general(v, oh_ref[...], (((1,), (0,)), ((), ())),
                            preferred_element_type=jnp.float32)

    px = _sig(g[0:1])
    py = _sig(g[1:2])
    pw = g[2:3]
    ph = g[3:4]
    pc = _clip(_sig(g[4:5]))
    bce0_pc = -jnp.log(1.0 - pc)
    bce1_pc = -jnp.log(pc)

    it0_tt = jax.lax.broadcasted_iota(jnp.int32, (_NT, _NT), 0)
    it1_tt = jax.lax.broadcasted_iota(jnp.int32, (_NT, _NT), 1)
    lower = it0_tt < it1_tt
    upper = it0_tt > it1_tt
    same_cell = key_col == key_row

    zer_row = (bn_row == a_f) | (ioua_row > _IGNORE_THRES)
    zer_col = (bn_col == a_f) | (ioua_col > _IGNORE_THRES)
    dup = jnp.sum((same_cell & lower & zer_col).astype(jnp.float32),
                  axis=0, keepdims=True) > 0.0
    mask_noobj = (zer_row & (~dup) & act_row).astype(jnp.float32)
    noobj_sub = jnp.sum(bce0_pc * mask_noobj)

    is_obj_row = (bn_row == a_f) & act_row
    is_obj_col = bn_col == a_f
    lose = jnp.sum((same_cell & upper & is_obj_col).astype(jnp.float32),
                   axis=0, keepdims=True) > 0.0
    win = (is_obj_row & (~lose)).astype(jnp.float32)

    saw_a = _scalar_pick(sa_w, a)
    sah_a = _scalar_pick(sa_h, a)
    tx = cx_row - jnp.floor(cx_row)
    ty = cy_row - jnp.floor(cy_row)
    tw = jnp.log(gw_row / saw_a + 1e-16)
    th = jnp.log(gh_row / sah_a + 1e-16)
    sq = (px - tx) ** 2 + (py - ty) ** 2 + (pw - tw) ** 2 + (ph - th) ** 2

    pcls = _clip(_sig(g[5:85]))
    cls0 = jnp.sum(-jnp.log(1.0 - pcls), axis=0, keepdims=True)
    itc = jax.lax.broadcasted_iota(jnp.int32, (_NUM_CLASSES, _NT), 0)
    oh_lab = (itc.astype(jnp.float32) == lab_row).astype(jnp.float32)
    p_lab = _clip(jnp.sum(pcls * oh_lab, axis=0, keepdims=True))

    lab_eq = lab_col == lab_row
    dup_lab = jnp.sum((same_cell & lower & is_obj_col & lab_eq)
                      .astype(jnp.float32), axis=0, keepdims=True) > 0.0
    mask_lab = (is_obj_row & (~dup_lab)).astype(jnp.float32)

    obj_add = jnp.sum(win * (sq + _OBJ_SCALE * bce1_pc + cls0))
    lab_add = jnp.sum(mask_lab * (-jnp.log(p_lab) + jnp.log(1.0 - p_lab)))

    delta = (_NOOBJ_SCALE * (base - noobj_sub) + obj_add + lab_add)

    @pl.when((b == 0) & (a == 0))
    def _init():
        loss_ref[0, 0] = 0.0

    loss_ref[0, 0] += delta


@jax.jit
def kernel(x, target):
    x4 = x.reshape(_B, _NUM_ANCHORS, _C, _S)
    tgt = target
    tgtT = target.T

    out4, loss = pl.pallas_call(
        _body,
        grid=(_B, _NUM_ANCHORS),
        in_specs=[
            pl.BlockSpec((1, 1, _C, _S), lambda b, a: (b, a, 0, 0)),
            pl.BlockSpec((_NT, 6), lambda b, a: (0, 0)),
            pl.BlockSpec((6, _NT), lambda b, a: (0, 0)),
        ],
        out_specs=[
            pl.BlockSpec((1, 1, _S, _C), lambda b, a: (b, a, 0, 0)),
            pl.BlockSpec((1, 1), lambda b, a: (0, 0),
                         memory_space=pltpu.SMEM),
        ],
        out_shape=[
            jax.ShapeDtypeStruct((_B, _NUM_ANCHORS, _S, _C), jnp.float32),
            jax.ShapeDtypeStruct((1, 1), jnp.float32),
        ],
        scratch_shapes=[pltpu.VMEM((_S, _NT), jnp.float32)],
        compiler_params=pltpu.CompilerParams(
            dimension_semantics=("arbitrary", "arbitrary")),
    )(x4, tgt, tgtT)

    output = out4.reshape(_B, _NUM_ANCHORS * _S, _C)
    return output, loss.reshape(())
